# baseline (speedup 1.0000x reference)
import functools
import jax
import jax.numpy as jnp
from jax import lax
from jax.experimental import pallas as pl
from jax.experimental.pallas import tpu as pltpu
from jax.experimental.pallas import tpu_sc as plsc

R = 2048
N = 32768
NW = 32
L = 16
STEPS = N // L
R_SC = 512


ACC = 4


def _row_argmax(buf):
    iota = lax.broadcasted_iota(jnp.int32, (L,), 0)

    def body(j, carry):
        ms, idxs, curs = carry
        base = j * (L * ACC)
        new_ms, new_idxs, new_curs = [], [], []
        for a in range(ACC):
            v = buf[pl.ds(base + a * L, L)]
            pred = v > ms[a]
            new_ms.append(jnp.where(pred, v, ms[a]))
            new_idxs.append(jnp.where(pred, curs[a], idxs[a]))
            new_curs.append(curs[a] + L * ACC)
        return tuple(new_ms), tuple(new_idxs), tuple(new_curs)

    m0 = jnp.full((L,), -jnp.inf, jnp.float32)
    init = (
        (m0,) * ACC,
        (iota * 0,) * ACC,
        tuple(iota + a * L for a in range(ACC)),
    )
    ms, idxs, _ = lax.fori_loop(0, STEPS // ACC, body, init, unroll=4)
    mall = ms[0]
    for a in range(1, ACC):
        mall = jnp.maximum(mall, ms[a])
    M = jnp.max(mall)
    big = jnp.int32(2**30)
    best = jnp.full((L,), big, jnp.int32)
    for a in range(ACC):
        best = jnp.minimum(best, jnp.where(ms[a] == M, idxs[a], big))
    return jnp.min(best)


def _insert(rvec, lane, val):
    iota = lax.broadcasted_iota(jnp.int32, (L,), 0)
    return jnp.where(iota == lane, val, rvec)


def _make_sc_argmax(rows):
    rpw = rows // NW
    res_n = ((rpw + L - 1) // L) * L
    mesh = plsc.VectorSubcoreMesh(core_axis_name="c", subcore_axis_name="s")

    @functools.partial(
        pl.kernel,
        mesh=mesh,
        compiler_params=pltpu.CompilerParams(needs_layout_passes=False),
        out_type=jax.ShapeDtypeStruct((rows,), jnp.int32),
        scratch_types=[
            pltpu.VMEM((N,), jnp.float32),
            pltpu.VMEM((N,), jnp.float32),
            pltpu.VMEM((res_n,), jnp.int32),
            pltpu.SemaphoreType.DMA,
            pltpu.SemaphoreType.DMA,
        ],
    )
    def sc_argmax(x_hbm, out_hbm, buf_a, buf_b, res, sem_a, sem_b):
        wid = lax.axis_index("s") * 2 + lax.axis_index("c")
        base = wid * rpw

        pltpu.async_copy(x_hbm.at[base], buf_a, sem_a)
        pltpu.async_copy(x_hbm.at[base + 1], buf_b, sem_b)

        def pair(g, rvec):
            r0 = 2 * g
            pltpu.make_async_copy(x_hbm.at[base], buf_a, sem_a).wait()
            i0 = _row_argmax(buf_a)
            pltpu.async_copy(x_hbm.at[base + r0 + 2], buf_a, sem_a)
            rvec = _insert(rvec, r0 & (L - 1), i0)
            pltpu.make_async_copy(x_hbm.at[base], buf_b, sem_b).wait()
            i1 = _row_argmax(buf_b)
            pltpu.async_copy(x_hbm.at[base + r0 + 3], buf_b, sem_b)
            rvec = _insert(rvec, (r0 + 1) & (L - 1), i1)

            @pl.when((g & 7) == 7)
            def _flush():
                res[pl.ds((g // 8) * L, L)] = rvec

            return rvec

        rvec = jnp.zeros((L,), jnp.int32)
        rvec = lax.fori_loop(0, rpw // 2 - 1, pair, rvec)
        pltpu.make_async_copy(x_hbm.at[base], buf_a, sem_a).wait()
        rvec = _insert(rvec, (rpw - 2) & (L - 1), _row_argmax(buf_a))
        pltpu.make_async_copy(x_hbm.at[base], buf_b, sem_b).wait()
        rvec = _insert(rvec, (rpw - 1) & (L - 1), _row_argmax(buf_b))
        res[pl.ds(((rpw - 1) // L) * L, L)] = rvec

        pltpu.sync_copy(res.at[pl.ds(0, rpw)], out_hbm.at[pl.ds(base, rpw)])

    return sc_argmax


TC_BB = 2


def _tc_body(x_ref, o_ref):
    xb = x_ref[...].reshape(TC_BB * 32, -1)
    m = jnp.max(xb, axis=-1, keepdims=True)
    iota = lax.broadcasted_iota(jnp.int32, xb.shape, 1)
    big = jnp.int32(jnp.iinfo(jnp.int32).max)
    idx = jnp.min(jnp.where(xb == m, iota, big), axis=-1)
    o_ref[...] = idx.reshape(TC_BB, 1, 32)


def _tc_argmax_tail(x3, skip):
    b, h, n = x3.shape
    nb = (b - skip) // TC_BB
    out = pl.pallas_call(
        _tc_body,
        grid=(nb,),
        in_specs=[pl.BlockSpec((TC_BB, h, n), lambda i: (i + skip // TC_BB, 0, 0))],
        out_specs=pl.BlockSpec((TC_BB, 1, h), lambda i: (i, 0, 0)),
        out_shape=jax.ShapeDtypeStruct((b - skip, 1, h), jnp.int32),
    )(x3)
    return out.reshape((b - skip) * h)


def kernel(x):
    B, H, n = x.shape
    flat = x.reshape(R, N)
    sc_out = _make_sc_argmax(R_SC)(flat)
    tc_out = _tc_argmax_tail(x, R_SC // 32)
    out = jnp.concatenate([sc_out, tc_out])
    return out.reshape(B, H).astype(jnp.int64)


# speedup vs baseline: 1.0189x; 1.0189x over previous
import functools
import jax
import jax.numpy as jnp
from jax import lax
from jax.experimental import pallas as pl
from jax.experimental.pallas import tpu as pltpu
from jax.experimental.pallas import tpu_sc as plsc

R = 2048
N = 32768
NW = 32
L = 16
STEPS = N // L
R_SC = 768


ACC = 4


def _row_argmax(buf):
    iota = lax.broadcasted_iota(jnp.int32, (L,), 0)

    def body(j, carry):
        ms, idxs, curs = carry
        base = j * (L * ACC)
        new_ms, new_idxs, new_curs = [], [], []
        for a in range(ACC):
            v = buf[pl.ds(base + a * L, L)]
            pred = v > ms[a]
            new_ms.append(jnp.where(pred, v, ms[a]))
            new_idxs.append(jnp.where(pred, curs[a], idxs[a]))
            new_curs.append(curs[a] + L * ACC)
        return tuple(new_ms), tuple(new_idxs), tuple(new_curs)

    m0 = jnp.full((L,), -jnp.inf, jnp.float32)
    init = (
        (m0,) * ACC,
        (iota * 0,) * ACC,
        tuple(iota + a * L for a in range(ACC)),
    )
    ms, idxs, _ = lax.fori_loop(0, STEPS // ACC, body, init, unroll=4)
    mall = ms[0]
    for a in range(1, ACC):
        mall = jnp.maximum(mall, ms[a])
    M = jnp.max(mall)
    big = jnp.int32(2**30)
    best = jnp.full((L,), big, jnp.int32)
    for a in range(ACC):
        best = jnp.minimum(best, jnp.where(ms[a] == M, idxs[a], big))
    return jnp.min(best)


def _insert(rvec, lane, val):
    iota = lax.broadcasted_iota(jnp.int32, (L,), 0)
    return jnp.where(iota == lane, val, rvec)


def _make_sc_argmax(rows):
    rpw = rows // NW
    res_n = ((rpw + L - 1) // L) * L
    mesh = plsc.VectorSubcoreMesh(core_axis_name="c", subcore_axis_name="s")

    @functools.partial(
        pl.kernel,
        mesh=mesh,
        compiler_params=pltpu.CompilerParams(needs_layout_passes=False),
        out_type=jax.ShapeDtypeStruct((rows,), jnp.int32),
        scratch_types=[
            pltpu.VMEM((N,), jnp.float32),
            pltpu.VMEM((N,), jnp.float32),
            pltpu.VMEM((res_n,), jnp.int32),
            pltpu.SemaphoreType.DMA,
            pltpu.SemaphoreType.DMA,
        ],
    )
    def sc_argmax(x_hbm, out_hbm, buf_a, buf_b, res, sem_a, sem_b):
        wid = lax.axis_index("s") * 2 + lax.axis_index("c")
        base = wid * rpw

        pltpu.async_copy(x_hbm.at[base], buf_a, sem_a)
        pltpu.async_copy(x_hbm.at[base + 1], buf_b, sem_b)

        def pair(g, rvec):
            r0 = 2 * g
            pltpu.make_async_copy(x_hbm.at[base], buf_a, sem_a).wait()
            i0 = _row_argmax(buf_a)
            pltpu.async_copy(x_hbm.at[base + r0 + 2], buf_a, sem_a)
            rvec = _insert(rvec, r0 & (L - 1), i0)
            pltpu.make_async_copy(x_hbm.at[base], buf_b, sem_b).wait()
            i1 = _row_argmax(buf_b)
            pltpu.async_copy(x_hbm.at[base + r0 + 3], buf_b, sem_b)
            rvec = _insert(rvec, (r0 + 1) & (L - 1), i1)

            @pl.when((g & 7) == 7)
            def _flush():
                res[pl.ds((g // 8) * L, L)] = rvec

            return rvec

        rvec = jnp.zeros((L,), jnp.int32)
        rvec = lax.fori_loop(0, rpw // 2 - 1, pair, rvec)
        pltpu.make_async_copy(x_hbm.at[base], buf_a, sem_a).wait()
        rvec = _insert(rvec, (rpw - 2) & (L - 1), _row_argmax(buf_a))
        pltpu.make_async_copy(x_hbm.at[base], buf_b, sem_b).wait()
        rvec = _insert(rvec, (rpw - 1) & (L - 1), _row_argmax(buf_b))
        res[pl.ds(((rpw - 1) // L) * L, L)] = rvec

        pltpu.sync_copy(res.at[pl.ds(0, rpw)], out_hbm.at[pl.ds(base, rpw)])

    return sc_argmax


TC_BB = 4


def _tc_body(x_ref, o_ref):
    xb = x_ref[...].reshape(TC_BB * 32, -1)
    m = jnp.max(xb, axis=-1, keepdims=True)
    iota = lax.broadcasted_iota(jnp.int32, xb.shape, 1)
    big = jnp.int32(jnp.iinfo(jnp.int32).max)
    idx = jnp.min(jnp.where(xb == m, iota, big), axis=-1)
    o_ref[...] = idx.reshape(TC_BB, 1, 32)


def _tc_argmax_tail(x3, skip):
    b, h, n = x3.shape
    nb = (b - skip) // TC_BB
    out = pl.pallas_call(
        _tc_body,
        grid=(nb,),
        in_specs=[pl.BlockSpec((TC_BB, h, n), lambda i: (i + skip // TC_BB, 0, 0))],
        out_specs=pl.BlockSpec((TC_BB, 1, h), lambda i: (i, 0, 0)),
        out_shape=jax.ShapeDtypeStruct((b - skip, 1, h), jnp.int32),
    )(x3)
    return out.reshape((b - skip) * h)


def kernel(x):
    B, H, n = x.shape
    flat = x.reshape(R, N)
    sc_out = _make_sc_argmax(R_SC)(flat)
    tc_out = _tc_argmax_tail(x, R_SC // 32)
    out = jnp.concatenate([sc_out, tc_out])
    return out.reshape(B, H).astype(jnp.int64)


# speedup vs baseline: 1.0220x; 1.0030x over previous
import functools
import jax
import jax.numpy as jnp
from jax import lax
from jax.experimental import pallas as pl
from jax.experimental.pallas import tpu as pltpu
from jax.experimental.pallas import tpu_sc as plsc

R = 2048
N = 32768
NW = 32
L = 16
STEPS = N // L
R_SC = 768


ACC = 4


def _row_argmax(buf):
    iota = lax.broadcasted_iota(jnp.int32, (L,), 0)

    def body(j, carry):
        ms, idxs, curs = carry
        base = j * (L * ACC)
        new_ms, new_idxs, new_curs = [], [], []
        for a in range(ACC):
            v = buf[pl.ds(base + a * L, L)]
            pred = v > ms[a]
            new_ms.append(jnp.where(pred, v, ms[a]))
            new_idxs.append(jnp.where(pred, curs[a], idxs[a]))
            new_curs.append(curs[a] + L * ACC)
        return tuple(new_ms), tuple(new_idxs), tuple(new_curs)

    m0 = jnp.full((L,), -jnp.inf, jnp.float32)
    init = (
        (m0,) * ACC,
        (iota * 0,) * ACC,
        tuple(iota + a * L for a in range(ACC)),
    )
    ms, idxs, _ = lax.fori_loop(0, STEPS // ACC, body, init, unroll=4)
    mall = ms[0]
    for a in range(1, ACC):
        mall = jnp.maximum(mall, ms[a])
    M = jnp.max(mall)
    big = jnp.int32(2**30)
    best = jnp.full((L,), big, jnp.int32)
    for a in range(ACC):
        best = jnp.minimum(best, jnp.where(ms[a] == M, idxs[a], big))
    return jnp.min(best)


def _insert(rvec, lane, val):
    iota = lax.broadcasted_iota(jnp.int32, (L,), 0)
    return jnp.where(iota == lane, val, rvec)


def _make_sc_argmax(rows):
    rpw = rows // NW
    res_n = ((rpw + L - 1) // L) * L
    mesh = plsc.VectorSubcoreMesh(core_axis_name="c", subcore_axis_name="s")

    @functools.partial(
        pl.kernel,
        mesh=mesh,
        compiler_params=pltpu.CompilerParams(needs_layout_passes=False),
        out_type=jax.ShapeDtypeStruct((rows,), jnp.int32),
        scratch_types=[
            pltpu.VMEM((N,), jnp.float32),
            pltpu.VMEM((N,), jnp.float32),
            pltpu.VMEM((res_n,), jnp.int32),
            pltpu.SemaphoreType.DMA,
            pltpu.SemaphoreType.DMA,
        ],
    )
    def sc_argmax(x_hbm, out_hbm, buf_a, buf_b, res, sem_a, sem_b):
        wid = lax.axis_index("s") * 2 + lax.axis_index("c")
        base = wid * rpw

        pltpu.async_copy(x_hbm.at[base], buf_a, sem_a)
        pltpu.async_copy(x_hbm.at[base + 1], buf_b, sem_b)

        def pair(g, rvec):
            r0 = 2 * g
            pltpu.make_async_copy(x_hbm.at[base], buf_a, sem_a).wait()
            i0 = _row_argmax(buf_a)
            pltpu.async_copy(x_hbm.at[base + r0 + 2], buf_a, sem_a)
            rvec = _insert(rvec, r0 & (L - 1), i0)
            pltpu.make_async_copy(x_hbm.at[base], buf_b, sem_b).wait()
            i1 = _row_argmax(buf_b)
            pltpu.async_copy(x_hbm.at[base + r0 + 3], buf_b, sem_b)
            rvec = _insert(rvec, (r0 + 1) & (L - 1), i1)

            @pl.when((g & 7) == 7)
            def _flush():
                res[pl.ds((g // 8) * L, L)] = rvec

            return rvec

        rvec = jnp.zeros((L,), jnp.int32)
        rvec = lax.fori_loop(0, rpw // 2 - 1, pair, rvec)
        pltpu.make_async_copy(x_hbm.at[base], buf_a, sem_a).wait()
        rvec = _insert(rvec, (rpw - 2) & (L - 1), _row_argmax(buf_a))
        pltpu.make_async_copy(x_hbm.at[base], buf_b, sem_b).wait()
        rvec = _insert(rvec, (rpw - 1) & (L - 1), _row_argmax(buf_b))
        res[pl.ds(((rpw - 1) // L) * L, L)] = rvec

        pltpu.sync_copy(res.at[pl.ds(0, rpw)], out_hbm.at[pl.ds(base, rpw)])

    return sc_argmax


TC_BB = 2


def _tc_body(x_ref, o_ref):
    xb = x_ref[...].reshape(TC_BB * 32, -1)
    m = jnp.max(xb, axis=-1, keepdims=True)
    iota = lax.broadcasted_iota(jnp.int32, xb.shape, 1)
    big = jnp.int32(jnp.iinfo(jnp.int32).max)
    idx = jnp.min(jnp.where(xb == m, iota, big), axis=-1)
    o_ref[...] = idx.reshape(TC_BB, 1, 32)


def _tc_argmax_tail(x3, skip):
    b, h, n = x3.shape
    nb = (b - skip) // TC_BB
    out = pl.pallas_call(
        _tc_body,
        grid=(nb,),
        in_specs=[pl.BlockSpec((TC_BB, h, n), lambda i: (i + skip // TC_BB, 0, 0))],
        out_specs=pl.BlockSpec((TC_BB, 1, h), lambda i: (i, 0, 0)),
        out_shape=jax.ShapeDtypeStruct((b - skip, 1, h), jnp.int32),
    )(x3)
    return out.reshape((b - skip) * h)


def kernel(x):
    B, H, n = x.shape
    flat = x.reshape(R, N)
    sc_out = _make_sc_argmax(R_SC)(flat)
    tc_out = _tc_argmax_tail(x, R_SC // 32)
    out = jnp.concatenate([sc_out, tc_out])
    return out.reshape(B, H).astype(jnp.int64)
